# baseline (speedup 1.0000x reference)
import functools

import jax
import jax.numpy as jnp
from jax.experimental import pallas as pl

_BF = jnp.bfloat16


def _fused_body(x_ref, *refs, ndim):
    out_ref = refs[-1]
    x = x_ref[...]
    acc = None
    for d in range(ndim):
        w1, b1, w2, b2, w3, b3, w4c = refs[7 * d:7 * d + 7]
        col = x[:, d:d + 1]
        h = jnp.maximum(col * w1[...] + b1[...], 0.0)
        h = jnp.dot(h.astype(_BF), w2[...], preferred_element_type=jnp.float32)
        h = jnp.maximum(h + b2[...], 0.0)
        h = jnp.dot(h.astype(_BF), w3[...], preferred_element_type=jnp.float32)
        h = jnp.maximum(h + b3[...], 0.0)
        g = jnp.dot(h.astype(_BF), w4c[...], preferred_element_type=jnp.float32)
        acc = g if acc is None else acc + g
    bc, wr2, br2, wr3, br3, wr4, br4 = refs[7 * ndim:7 * ndim + 7]
    h = jnp.maximum(acc + bc[...], 0.0)
    h = jnp.dot(h.astype(_BF), wr2[...], preferred_element_type=jnp.float32)
    h = jnp.maximum(h + br2[...], 0.0)
    h = jnp.dot(h.astype(_BF), wr3[...], preferred_element_type=jnp.float32)
    h = jnp.maximum(h + br3[...], 0.0)
    out_ref[...] = (
        jnp.dot(h.astype(_BF), wr4[...], preferred_element_type=jnp.float32)
        + br4[...])


def kernel(x, nets, rho_params):
    n, ndim = x.shape
    zdim = rho_params[-1][0].shape[1]

    wr1, br1 = rho_params[0]

    args = []
    bc_terms = br1
    for net in nets:
        (w1, b1), (w2, b2), (w3, b3), (w4, b4) = net
        w4c = (w4 @ wr1).astype(_BF)
        bc_terms = bc_terms + b4 @ wr1
        args += [w1, b1[None, :], w2.astype(_BF), b2[None, :],
                 w3.astype(_BF), b3[None, :], w4c]
    args.append(bc_terms[None, :])
    for (w, b) in rho_params[1:]:
        args += [w.astype(_BF), b[None, :]]

    blk = 1024
    n_pad = ((n + blk - 1) // blk) * blk
    xp = x if n_pad == n else jnp.pad(x, ((0, n_pad - n), (0, 0)))

    full = lambda a: pl.BlockSpec(a.shape, lambda i: (0,) * a.ndim)
    out = pl.pallas_call(
        functools.partial(_fused_body, ndim=ndim),
        grid=(n_pad // blk,),
        in_specs=[pl.BlockSpec((blk, ndim), lambda i: (i, 0))]
                 + [full(a) for a in args],
        out_specs=pl.BlockSpec((blk, zdim), lambda i: (i, 0)),
        out_shape=jax.ShapeDtypeStruct((n_pad, zdim), jnp.float32),
    )(xp, *args)
    return out[:n] if n_pad != n else out
